# baseline (speedup 1.0000x reference)
import functools

import jax
import jax.numpy as jnp
from jax.experimental import pallas as pl
from jax.experimental.pallas import tpu as pltpu

B = 16384
N_IN = 256
SIZES = (128, 128, 128, 128, 64)
TOTALS = (256, 384, 512, 640, 768)
K = 16
BLK = 4096
OFF = (0, 128, 256, 384, 512, 576)


def _body(idx0, idx1, idx2, idx3, idx4, w0, w1, w2, w3, w4, x_ref,
          out_ref, wx, w0a, w01, w2a, w23):
    idx_refs = (idx0, idx1, idx2, idx3, idx4)
    w_refs = (w0, w1, w2, w3, w4)

    @pl.when(pl.program_id(0) == 0)
    def _densify():
        for li in range(5):
            sz = SIZES[li]
            rows = TOTALS[li]
            idx = idx_refs[li][...]
            idx = jnp.where(idx < N_IN, N_IN - 1 - idx, idx)
            w = w_refs[li][...]
            row_id = jax.lax.broadcasted_iota(jnp.int32, (rows, sz), 0)
            m = jnp.zeros((rows, sz), dtype=jnp.float32)
            for k in range(K):
                m = m + jnp.where(row_id == idx[k][None, :],
                                  w[k][None, :], 0.0)
            c0, c1 = OFF[li], OFF[li] + sz
            wx[:, c0:c1] = m[:256]
            if li == 1:
                w0a[...] = m[256:384]
            if li >= 2:
                w01[0:128, c0 - 256:c1 - 256] = m[256:384]
                w01[128:256, c0 - 256:c1 - 256] = (
                    m[384:512] if rows > 384
                    else jnp.zeros((128, sz), jnp.float32))
            if li == 3:
                w2a[...] = m[512:640]
            if li == 4:
                w23[0:128, :] = m[512:640]
                w23[128:256, :] = m[640:768]

    dot = functools.partial(jnp.dot, preferred_element_type=jnp.float32)
    half = BLK // 2
    for c in range(2):
        rows = pl.ds(c * half, half)
        x = x_ref[rows, :]
        X = dot(x, wx[...])
        h0 = jnp.tanh(X[:, 0:128])
        h1 = jax.nn.relu(X[:, 128:256] + dot(h0, w0a[...]))
        T = dot(jnp.concatenate([h0, h1], axis=1), w01[...])
        h2 = jax.nn.sigmoid(X[:, 256:384] + T[:, 0:128])
        h3 = jnp.tanh(X[:, 384:512] + T[:, 128:256] + dot(h2, w2a[...]))
        out_ref[rows, :] = (X[:, 512:576] + T[:, 256:320]
                            + dot(jnp.concatenate([h2, h3], axis=1),
                                  w23[...]))


def kernel(x, idx0, idx1, idx2, idx3, idx4, w0, w1, w2, w3, w4):
    idxs = [a.T for a in (idx0, idx1, idx2, idx3, idx4)]
    ws = [a.T for a in (w0, w1, w2, w3, w4)]

    grid = (B // BLK,)
    out = pl.pallas_call(
        _body,
        grid=grid,
        in_specs=[pl.BlockSpec((K, SIZES[li]), lambda i: (0, 0))
                  for li in range(5)] * 2
        + [pl.BlockSpec((BLK, N_IN), lambda i: (i, 0))],
        out_specs=pl.BlockSpec((BLK, SIZES[-1]), lambda i: (i, 0)),
        out_shape=jax.ShapeDtypeStruct((B, SIZES[-1]), jnp.float32),
        scratch_shapes=[
            pltpu.VMEM((256, 576), jnp.float32),
            pltpu.VMEM((128, 128), jnp.float32),
            pltpu.VMEM((256, 320), jnp.float32),
            pltpu.VMEM((128, 128), jnp.float32),
            pltpu.VMEM((256, 64), jnp.float32),
        ],
    )(*idxs, *ws, x)
    return out
